# Initial kernel scaffold; baseline (speedup 1.0000x reference)
import jax
import jax.numpy as jnp
from jax.experimental import pallas as pl


def kernel(h, edge_index, r, norm, emb, basis1, comp1, loop1, bias1, basis2, comp2, loop2, bias2):
    raise NotImplementedError("write your pallas kernel here")


# speedup vs baseline: 8.2051x; 8.2051x over previous
import functools

import jax
import jax.numpy as jnp
from jax import lax
from jax.experimental import pallas as pl
from jax.experimental.pallas import tpu as pltpu
from jax.experimental.pallas import tpu_sc as plsc

N, E, D, R, B = 10000, 160000, 128, 64, 8
N_PAD = 10240
TE = 640
TN = 640
CHUNK = 128
NCHUNKS = E // CHUNK
NC, NS = 2, 16
NW = NC * NS
ZROWS = N_PAD // NS


def _worker_chunks(wid):
    base_n, extra = NCHUNKS // NW, NCHUNKS % NW
    return jnp.where(wid < extra, base_n + 1, base_n)


def _sc_gather(x_pad, src):
    mesh = plsc.VectorSubcoreMesh(core_axis_name="c", subcore_axis_name="s")

    @functools.partial(
        pl.kernel, mesh=mesh,
        out_type=jax.ShapeDtypeStruct((E, D), jnp.float32),
        scratch_types=[
            pltpu.VMEM((CHUNK,), jnp.int32),
            pltpu.VMEM((CHUNK, D), jnp.float32),
            pltpu.SemaphoreType.DMA,
        ],
    )
    def k(x_hbm, src_hbm, out_hbm, idx_v, rows_v, sem):
        c = lax.axis_index("c")
        s = lax.axis_index("s")
        wid = s * NC + c

        def body(j, carry):
            base = (wid + NW * j) * CHUNK
            pltpu.sync_copy(src_hbm.at[pl.ds(base, CHUNK)], idx_v)
            pltpu.async_copy(x_hbm.at[idx_v], rows_v, sem).wait()
            pltpu.sync_copy(rows_v, out_hbm.at[pl.ds(base, CHUNK)])
            return carry

        lax.fori_loop(0, _worker_chunks(wid), body, 0)

    return k(x_pad, src)


def _sc_scatter(msg, dst, zrows):
    mesh = plsc.VectorSubcoreMesh(core_axis_name="c", subcore_axis_name="s")

    @functools.partial(
        pl.kernel, mesh=mesh,
        out_type=jax.ShapeDtypeStruct((NC, N_PAD, D), jnp.float32),
        scratch_types=[
            pltpu.VMEM_SHARED((N_PAD, D), jnp.float32),
            pltpu.VMEM((CHUNK,), jnp.int32),
            pltpu.VMEM((CHUNK, D), jnp.float32),
            pltpu.SemaphoreType.DMA,
        ],
    )
    def k(msg_hbm, dst_hbm, zero_hbm, out_hbm, accum, idx_v, rows_v, sem):
        c = lax.axis_index("c")
        s = lax.axis_index("s")
        wid = s * NC + c
        pltpu.sync_copy(zero_hbm, accum.at[pl.ds(s * ZROWS, ZROWS)])
        plsc.subcore_barrier()

        def body(j, carry):
            base = (wid + NW * j) * CHUNK
            pltpu.sync_copy(dst_hbm.at[pl.ds(base, CHUNK)], idx_v)
            pltpu.sync_copy(msg_hbm.at[pl.ds(base, CHUNK)], rows_v)
            pltpu.sync_copy(rows_v, accum.at[idx_v], add=True)
            return carry

        lax.fori_loop(0, _worker_chunks(wid), body, 0)
        plsc.subcore_barrier()
        pltpu.sync_copy(accum.at[pl.ds(s * ZROWS, ZROWS)],
                        out_hbm.at[c, pl.ds(s * ZROWS, ZROWS)])

    return k(msg, dst, zrows)


def _msg_body(xs_ref, r_ref, norm_ref, basis_ref, comp_ref, out_ref):
    xt = xs_ref[:]
    rt = r_ref[0]
    nt = norm_ref[0]
    onehot_t = (rt == lax.broadcasted_iota(jnp.int32, (R, TE), 0))
    onehot_t = onehot_t.astype(jnp.float32) * nt
    coefs = lax.dot_general(onehot_t, comp_ref[:],
                            (((0,), (0,)), ((), ())),
                            preferred_element_type=jnp.float32)
    y = jnp.dot(xt, basis_ref[:], preferred_element_type=jnp.float32)
    acc = coefs[:, 0:1] * y[:, 0:D]
    for b in range(1, B):
        acc = acc + coefs[:, b:b + 1] * y[:, b * D:(b + 1) * D]
    out_ref[:] = acc


def _tc_msg(xs, r3, norm3, basis_flat, comp_pad):
    return pl.pallas_call(
        _msg_body,
        grid=(E // TE,),
        in_specs=[
            pl.BlockSpec((TE, D), lambda i: (i, 0)),
            pl.BlockSpec((1, 1, TE), lambda i: (i, 0, 0)),
            pl.BlockSpec((1, 1, TE), lambda i: (i, 0, 0)),
            pl.BlockSpec((D, B * D), lambda i: (0, 0)),
            pl.BlockSpec((R, 128), lambda i: (0, 0)),
        ],
        out_specs=pl.BlockSpec((TE, D), lambda i: (i, 0)),
        out_shape=jax.ShapeDtypeStruct((E, D), jnp.float32),
    )(xs, r3, norm3, basis_flat, comp_pad)


def _final_body(act, parts_ref, x_ref, loop_ref, bias_ref, out_ref):
    pre = (parts_ref[0] + parts_ref[1]
           + jnp.dot(x_ref[:], loop_ref[:], preferred_element_type=jnp.float32)
           + bias_ref[:])
    out_ref[:] = act(pre)


def _tc_final(parts, x_pad, loop_w, bias2d, act):
    return pl.pallas_call(
        functools.partial(_final_body, act),
        grid=(N_PAD // TN,),
        in_specs=[
            pl.BlockSpec((NC, TN, D), lambda i: (0, i, 0)),
            pl.BlockSpec((TN, D), lambda i: (i, 0)),
            pl.BlockSpec((D, D), lambda i: (0, 0)),
            pl.BlockSpec((1, D), lambda i: (0, 0)),
        ],
        out_specs=pl.BlockSpec((TN, D), lambda i: (i, 0)),
        out_shape=jax.ShapeDtypeStruct((N_PAD, D), jnp.float32),
    )(parts, x_pad, loop_w, bias2d)


def _layer(x_pad, src, dst, r3, norm3, zrows, basis, comp, loop_w, bias, act):
    basis_flat = jnp.transpose(basis, (1, 0, 2)).reshape(D, B * D)
    comp_pad = jnp.pad(comp, ((0, 0), (0, 128 - B)))
    xs = _sc_gather(x_pad, src)
    msg = _tc_msg(xs, r3, norm3, basis_flat, comp_pad)
    parts = _sc_scatter(msg, dst, zrows)
    return _tc_final(parts, x_pad, loop_w, bias.reshape(1, D), act)


def kernel(h, edge_index, r, norm, emb, basis1, comp1, loop1, bias1,
           basis2, comp2, loop2, bias2):
    src = edge_index[0]
    dst = edge_index[1]
    x = jnp.take(emb, h, axis=0)
    x_pad = jnp.pad(x, ((0, N_PAD - N), (0, 0)))
    r3 = r.reshape(E // TE, 1, TE)
    norm3 = norm.reshape(E // TE, 1, TE)
    zrows = jnp.zeros((ZROWS, D), jnp.float32)
    x_pad = _layer(x_pad, src, dst, r3, norm3, zrows,
                   basis1, comp1, loop1, bias1, jax.nn.relu)
    x_pad = _layer(x_pad, src, dst, r3, norm3, zrows,
                   basis2, comp2, loop2, bias2, jax.nn.sigmoid)
    return x_pad[:N]


# speedup vs baseline: 11.8550x; 1.4448x over previous
import functools

import jax
import jax.numpy as jnp
from jax import lax
from jax.experimental import pallas as pl
from jax.experimental.pallas import tpu as pltpu
from jax.experimental.pallas import tpu_sc as plsc

N, E, D, R, B = 10000, 160000, 128, 64, 8
N_PAD = 10240
EH = E // 2
TE = 1600
TN = 640
CHUNK = 128
NC, NS = 2, 16
NW = NC * NS
ZROWS = N_PAD // NS


def _worker_chunks(wid, nchunks):
    base_n, extra = nchunks // NW, nchunks % NW
    return jnp.where(wid < extra, base_n + 1, base_n)


def _sc_gather(x_pad, src):
    ne = src.shape[0]
    mesh = plsc.VectorSubcoreMesh(core_axis_name="c", subcore_axis_name="s")

    @functools.partial(
        pl.kernel, mesh=mesh,
        out_type=jax.ShapeDtypeStruct((ne, D), jnp.float32),
        scratch_types=[
            pltpu.VMEM((CHUNK,), jnp.int32),
            pltpu.VMEM((CHUNK, D), jnp.float32),
            pltpu.SemaphoreType.DMA,
        ],
    )
    def k(x_hbm, src_hbm, out_hbm, idx_v, rows_v, sem):
        c = lax.axis_index("c")
        s = lax.axis_index("s")
        wid = s * NC + c

        def body(j, carry):
            base = (wid + NW * j) * CHUNK
            pltpu.sync_copy(src_hbm.at[pl.ds(base, CHUNK)], idx_v)
            pltpu.async_copy(x_hbm.at[idx_v], rows_v, sem).wait()
            pltpu.sync_copy(rows_v, out_hbm.at[pl.ds(base, CHUNK)])
            return carry

        lax.fori_loop(0, _worker_chunks(wid, ne // CHUNK), body, 0)

    return k(x_pad, src)


def _sc_scatter(msg, dst, zrows):
    ne = dst.shape[0]
    mesh = plsc.VectorSubcoreMesh(core_axis_name="c", subcore_axis_name="s")

    @functools.partial(
        pl.kernel, mesh=mesh,
        out_type=jax.ShapeDtypeStruct((NC, N_PAD, D), jnp.float32),
        scratch_types=[
            pltpu.VMEM_SHARED((N_PAD, D), jnp.float32),
            pltpu.VMEM((CHUNK,), jnp.int32),
            pltpu.VMEM((CHUNK, D), jnp.float32),
            pltpu.SemaphoreType.DMA,
        ],
    )
    def k(msg_hbm, dst_hbm, zero_hbm, out_hbm, accum, idx_v, rows_v, sem):
        c = lax.axis_index("c")
        s = lax.axis_index("s")
        wid = s * NC + c
        pltpu.sync_copy(zero_hbm, accum.at[pl.ds(s * ZROWS, ZROWS)])
        plsc.subcore_barrier()

        def body(j, carry):
            base = (wid + NW * j) * CHUNK
            pltpu.sync_copy(dst_hbm.at[pl.ds(base, CHUNK)], idx_v)
            pltpu.sync_copy(msg_hbm.at[pl.ds(base, CHUNK)], rows_v)
            pltpu.sync_copy(rows_v, accum.at[idx_v], add=True)
            return carry

        lax.fori_loop(0, _worker_chunks(wid, ne // CHUNK), body, 0)
        plsc.subcore_barrier()
        pltpu.sync_copy(accum.at[pl.ds(s * ZROWS, ZROWS)],
                        out_hbm.at[c, pl.ds(s * ZROWS, ZROWS)])

    return k(msg, dst, zrows)


def _msg_body(xs_ref, r_ref, norm_ref, basis_ref, comp_ref, out_ref):
    xt = xs_ref[:].astype(jnp.bfloat16)
    rt = r_ref[0]
    nt = norm_ref[0]
    onehot_t = (rt == lax.broadcasted_iota(jnp.int32, (R, TE), 0))
    onehot_t = (onehot_t.astype(jnp.float32) * nt).astype(jnp.bfloat16)
    cw = lax.dot_general(onehot_t, comp_ref[:],
                         (((0,), (0,)), ((), ())),
                         preferred_element_type=jnp.float32).astype(jnp.bfloat16)
    xnw = jnp.concatenate([xt] * B, axis=1) * cw
    out_ref[:] = jnp.dot(xnw, basis_ref[:],
                         preferred_element_type=jnp.float32)


def _tc_msg(xs, r3, norm3, basis_flat, comp_wide):
    ne = xs.shape[0]
    return pl.pallas_call(
        _msg_body,
        grid=(ne // TE,),
        in_specs=[
            pl.BlockSpec((TE, D), lambda i: (i, 0)),
            pl.BlockSpec((1, 1, TE), lambda i: (i, 0, 0)),
            pl.BlockSpec((1, 1, TE), lambda i: (i, 0, 0)),
            pl.BlockSpec((B * D, D), lambda i: (0, 0)),
            pl.BlockSpec((R, B * D), lambda i: (0, 0)),
        ],
        out_specs=pl.BlockSpec((TE, D), lambda i: (i, 0)),
        out_shape=jax.ShapeDtypeStruct((ne, D), jnp.float32),
    )(xs, r3, norm3, basis_flat, comp_wide)


def _final_body(act, pa_ref, pb_ref, x_ref, loop_ref, bias_ref, out_ref):
    pre = (pa_ref[0] + pa_ref[1] + pb_ref[0] + pb_ref[1]
           + jnp.dot(x_ref[:], loop_ref[:], preferred_element_type=jnp.float32)
           + bias_ref[:])
    out_ref[:] = act(pre)


def _tc_final(parts_a, parts_b, x_pad, loop_w, bias2d, act):
    return pl.pallas_call(
        functools.partial(_final_body, act),
        grid=(N_PAD // TN,),
        in_specs=[
            pl.BlockSpec((NC, TN, D), lambda i: (0, i, 0)),
            pl.BlockSpec((NC, TN, D), lambda i: (0, i, 0)),
            pl.BlockSpec((TN, D), lambda i: (i, 0)),
            pl.BlockSpec((D, D), lambda i: (0, 0)),
            pl.BlockSpec((1, D), lambda i: (0, 0)),
        ],
        out_specs=pl.BlockSpec((TN, D), lambda i: (i, 0)),
        out_shape=jax.ShapeDtypeStruct((N_PAD, D), jnp.float32),
    )(parts_a, parts_b, x_pad, loop_w, bias2d)


def _layer(x_pad, halves, zrows, basis, comp, loop_w, bias, act):
    basis_flat = basis.reshape(B * D, D).astype(jnp.bfloat16)
    comp_wide = jnp.repeat(comp, D, axis=1).astype(jnp.bfloat16)
    parts = []
    msgs = [None, None]
    for i, (src_h, dst_h, r3_h, n3_h) in enumerate(halves):
        xs = _sc_gather(x_pad, src_h)
        msgs[i] = _tc_msg(xs, r3_h, n3_h, basis_flat, comp_wide)
    for i, (src_h, dst_h, r3_h, n3_h) in enumerate(halves):
        parts.append(_sc_scatter(msgs[i], dst_h, zrows))
    return _tc_final(parts[0], parts[1], x_pad, loop_w,
                     bias.reshape(1, D), act)


def kernel(h, edge_index, r, norm, emb, basis1, comp1, loop1, bias1,
           basis2, comp2, loop2, bias2):
    src = edge_index[0]
    dst = edge_index[1]
    x = jnp.take(emb, h, axis=0)
    x_pad = jnp.pad(x, ((0, N_PAD - N), (0, 0)))
    halves = []
    for i in range(2):
        sl = slice(i * EH, (i + 1) * EH)
        halves.append((src[sl], dst[sl],
                       r[sl].reshape(EH // TE, 1, TE),
                       norm[sl].reshape(EH // TE, 1, TE)))
    zrows = jnp.zeros((ZROWS, D), jnp.float32)
    x_pad = _layer(x_pad, halves, zrows, basis1, comp1, loop1, bias1,
                   jax.nn.relu)
    x_pad = _layer(x_pad, halves, zrows, basis2, comp2, loop2, bias2,
                   jax.nn.sigmoid)
    return x_pad[:N]
